# baseline (speedup 1.0000x reference)
import functools

import jax
import jax.numpy as jnp
from jax import lax
from jax.experimental import pallas as pl
from jax.experimental.pallas import tpu as pltpu
from jax.experimental.pallas import tpu_sc as plsc


_B = 256
_NW = 32


def _main_body(nb, x_ref, gram_ref, norms_ref):
    i = pl.program_id(0)
    X = x_ref[...]
    g = jax.lax.dot_general(X, X, (((0,), (0,)), ((), ())),
                            preferred_element_type=jnp.float32)
    X3 = X.reshape(nb // 128, 128, 128)
    norms_ref[...] = jnp.sqrt(jnp.sum(X3 * X3, axis=2))[None]

    @pl.when(i == 0)
    def _init():
        gram_ref[...] = g

    @pl.when(i > 0)
    def _acc():
        gram_ref[...] += g


def _sc_body(chunk, norms_hbm, ids_hbm, psum_hbm, pcnt_hbm,
             norms_v, ids_v, acc_s, acc_c, fill_v):
    wid = lax.axis_index("s") * 2 + lax.axis_index("c")
    base = wid * chunk
    pltpu.sync_copy(norms_hbm.at[pl.ds(base, chunk)], norms_v)
    pltpu.sync_copy(ids_hbm.at[pl.ds(base, chunk)], ids_v.at[pl.ds(0, chunk)])
    ids_v[pl.ds(chunk, 16)] = jnp.full((16,), -1, jnp.int32)
    zero16 = jnp.zeros((16,), jnp.float32)
    for k in range(_B // 16):
        acc_s[pl.ds(16 * k, 16)] = zero16
        acc_c[pl.ds(16 * k, 16)] = zero16
    lane = lax.iota(jnp.int32, 16)

    def _step(i, carry):
        v = norms_v[pl.ds(i * 16, 16)]
        ids = ids_v[pl.ds(i * 16, 16)]
        nxt = ids_v[pl.ds(i * 16 + 1, 16)]
        boundary = ids != nxt
        prefix = plsc.cumsum(v) + carry
        posf = (i * 16 + lane + 1).astype(jnp.float32)
        plsc.store_scatter(acc_s, [ids], prefix, mask=boundary)
        plsc.store_scatter(acc_c, [ids], posf, mask=boundary)
        return carry + jnp.sum(v)

    lax.fori_loop(0, chunk // 16, _step, jnp.float32(0.0))

    for acc in (acc_s, acc_c):
        fill_v[pl.ds(0, 16)] = zero16
        carry = jnp.float32(0.0)
        for k in range(_B // 16):
            f = jnp.maximum(plsc.cummax(acc[pl.ds(16 * k, 16)]), carry)
            fill_v[pl.ds(16 + 16 * k, 16)] = f
            carry = jnp.max(f)
        for k in range(_B // 16):
            acc[pl.ds(16 * k, 16)] = (fill_v[pl.ds(16 + 16 * k, 16)]
                                      - fill_v[pl.ds(15 + 16 * k, 16)])

    pltpu.sync_copy(acc_s, psum_hbm.at[pl.ds(wid * _B, _B)])
    pltpu.sync_copy(acc_c, pcnt_hbm.at[pl.ds(wid * _B, _B)])


def _epilogue_body(g_ref, ps_ref, pc_ref, nm_ref, sr_ref):
    G0 = g_ref[...]
    row = jax.lax.broadcasted_iota(jnp.int32, (128, 128), 0)
    col = jax.lax.broadcasted_iota(jnp.int32, (128, 128), 1)
    fro2 = jnp.sum(jnp.where(row == col, G0, 0.0))

    Gm = G0
    for _ in range(20):
        s = jnp.max(jnp.abs(Gm))
        Gn = Gm / jnp.maximum(s, 1e-30)
        Gm = jax.lax.dot_general(Gn, Gn, (((1,), (0,)), ((), ())),
                                 preferred_element_type=jnp.float32)
    coln = jnp.sum(Gm * Gm, axis=0, keepdims=True)
    lane = jax.lax.broadcasted_iota(jnp.int32, (1, 128), 1)
    j = jnp.min(jnp.where(coln == jnp.max(coln), lane, 256))
    v_col = jnp.sum(jnp.where(col == j, Gm, 0.0), axis=1, keepdims=True)
    v_row = jnp.sum(jnp.where(row == j, Gm, 0.0), axis=0, keepdims=True)
    t_col = jnp.sum(G0 * v_row, axis=1, keepdims=True)
    sigma_max_sq = jnp.sum(t_col * v_col) / jnp.sum(v_col * v_col)
    sr_ref[0, 0] = fro2 / sigma_max_sq

    seg = jnp.sum(ps_ref[...], axis=0, keepdims=True)
    cnt = jnp.sum(pc_ref[...], axis=0, keepdims=True)
    per_graph = seg / jnp.maximum(cnt, 1.0)
    bidx = jax.lax.broadcasted_iota(jnp.int32, (1, _B), 1)
    bs = jnp.max(jnp.where(cnt > 0.0, bidx, -1))
    nm = jnp.sum(jnp.where(bidx < bs, per_graph, 0.0))
    nm_ref[0, 0] = nm / (bs + 1).astype(jnp.float32)


def kernel(input, batch):
    n, d = input.shape
    assert d == 128 and n % 128 == 0 and n % (16 * _NW) == 0
    nb = 6400 if n % 6400 == 0 else n
    grid = n // nb
    chunk = n // _NW

    gram, norms2d = pl.pallas_call(
        functools.partial(_main_body, nb),
        grid=(grid,),
        in_specs=[pl.BlockSpec((nb, d), lambda i: (i, 0))],
        out_specs=[
            pl.BlockSpec((d, d), lambda i: (0, 0)),
            pl.BlockSpec((1, nb // 128, 128), lambda i: (i, 0, 0)),
        ],
        out_shape=[
            jax.ShapeDtypeStruct((d, d), jnp.float32),
            jax.ShapeDtypeStruct((grid, nb // 128, 128), jnp.float32),
        ],
    )(input)

    sc_call = pl.kernel(
        functools.partial(_sc_body, chunk),
        out_type=[
            jax.ShapeDtypeStruct((_NW * _B,), jnp.float32),
            jax.ShapeDtypeStruct((_NW * _B,), jnp.float32),
        ],
        mesh=plsc.VectorSubcoreMesh(core_axis_name="c", subcore_axis_name="s",
                                    num_cores=2, num_subcores=16),
        compiler_params=pltpu.CompilerParams(needs_layout_passes=False),
        scratch_types=[
            pltpu.VMEM((chunk,), jnp.float32),
            pltpu.VMEM((chunk + 16,), jnp.int32),
            pltpu.VMEM((_B,), jnp.float32),
            pltpu.VMEM((_B,), jnp.float32),
            pltpu.VMEM((_B + 16,), jnp.float32),
        ],
    )
    psum, pcnt = sc_call(norms2d.reshape(n), batch.astype(jnp.int32))

    nm, sr = pl.pallas_call(
        _epilogue_body,
        in_specs=[
            pl.BlockSpec((d, d), lambda: (0, 0)),
            pl.BlockSpec((_NW, _B), lambda: (0, 0)),
            pl.BlockSpec((_NW, _B), lambda: (0, 0)),
        ],
        out_specs=[
            pl.BlockSpec((1, 1), lambda: (0, 0), memory_space=pltpu.SMEM),
            pl.BlockSpec((1, 1), lambda: (0, 0), memory_space=pltpu.SMEM),
        ],
        out_shape=[
            jax.ShapeDtypeStruct((1, 1), jnp.float32),
            jax.ShapeDtypeStruct((1, 1), jnp.float32),
        ],
    )(gram, psum.reshape(_NW, _B), pcnt.reshape(_NW, _B))

    return (input, nm[0, 0], sr[0, 0])
